# baseline (speedup 1.0000x reference)
import functools

import jax
import jax.numpy as jnp
from jax.experimental import pallas as pl

NUM_CLASSES = 80
CONF_T = 0.05
IOU_T = 0.5
MAX_PER_CLASS = 100
MAX_DET = 100

N_ANCHORS = 49104
CHUNK = 4464
N_CHUNKS = N_ANCHORS // CHUNK


def _decode_kernel(pred_ref, anc_ref, scores_ref, boxes_ref):
    p = pred_ref[0]
    anc = anc_ref[...]

    logits = p[:, 4:4 + NUM_CLASSES]
    s = jax.nn.sigmoid(logits)
    scores_ref[0] = jnp.where(s >= CONF_T, s, -1.0)

    bx = p[:, 0:1] * 0.1
    by = p[:, 1:2] * 0.1
    bw = p[:, 2:3] * 0.2
    bh = p[:, 3:4] * 0.2
    acx = anc[:, 0:1]
    acy = anc[:, 1:2]
    aw = anc[:, 2:3]
    ah = anc[:, 3:4]
    cx = bx * aw + acx
    cy = by * ah + acy
    w = jnp.exp(bw) * aw
    h = jnp.exp(bh) * ah
    boxes_ref[0] = jnp.concatenate(
        [cx - w / 2.0, cy - h / 2.0, cx + w / 2.0, cy + h / 2.0], axis=-1)


def _decode_stage(predictions, anchor_boxes):
    batch = predictions.shape[0]
    scores, boxes = pl.pallas_call(
        _decode_kernel,
        grid=(batch, N_CHUNKS),
        in_specs=[
            pl.BlockSpec((1, CHUNK, 4 + NUM_CLASSES), lambda b, j: (b, j, 0)),
            pl.BlockSpec((CHUNK, 4), lambda b, j: (j, 0)),
        ],
        out_specs=[
            pl.BlockSpec((1, CHUNK, NUM_CLASSES), lambda b, j: (b, j, 0)),
            pl.BlockSpec((1, CHUNK, 4), lambda b, j: (b, j, 0)),
        ],
        out_shape=[
            jax.ShapeDtypeStruct((batch, N_ANCHORS, NUM_CLASSES), jnp.float32),
            jax.ShapeDtypeStruct((batch, N_ANCHORS, 4), jnp.float32),
        ],
    )(predictions, anchor_boxes)
    return scores, boxes


def _nms_one_class(boxes_c, scores):
    top_s, top_i = jax.lax.top_k(scores, MAX_PER_CLASS)
    cand = boxes_c[top_i]
    ar = jnp.arange(MAX_PER_CLASS)

    def body(sc, i):
        valid = sc[i] > 0.0
        b = cand[i]
        x1 = jnp.maximum(b[0], cand[:, 0])
        y1 = jnp.maximum(b[1], cand[:, 1])
        x2 = jnp.minimum(b[2], cand[:, 2])
        y2 = jnp.minimum(b[3], cand[:, 3])
        inter = jnp.clip(x2 - x1, 0.0) * jnp.clip(y2 - y1, 0.0)
        a1 = (b[2] - b[0]) * (b[3] - b[1])
        a2 = (cand[:, 2] - cand[:, 0]) * (cand[:, 3] - cand[:, 1])
        iou = inter / (a1 + a2 - inter + 1e-8)
        suppress = (iou > IOU_T) & (ar > i) & valid
        return jnp.where(suppress, -1.0, sc), None

    s_final, _ = jax.lax.scan(body, top_s, ar)
    return cand, s_final


def _per_image(boxes_c, cls_scores):
    cand, s = jax.vmap(_nms_one_class, in_axes=(None, 1))(boxes_c, cls_scores)
    flat_s = s.reshape(-1)
    flat_b = cand.reshape(-1, 4)
    fs, fi = jax.lax.top_k(flat_s, MAX_DET)
    fb = flat_b[fi]
    cls_ids = fi // MAX_PER_CLASS
    valid = fs > 0.0
    cls_ids = jnp.where(valid, cls_ids, -1)
    num = jnp.sum(valid.astype(jnp.int32))
    return fb, fs, cls_ids, num


@functools.partial(jax.jit, static_argnums=())
def kernel(images, predictions, anchor_boxes):
    del images
    scores, boxes_c = _decode_stage(predictions, anchor_boxes)
    return jax.vmap(_per_image)(boxes_c, scores)


# speedup vs baseline: 7.4726x; 7.4726x over previous
import functools

import jax
import jax.numpy as jnp
import numpy as np
from jax.experimental import pallas as pl
from jax.experimental.pallas import tpu as pltpu

NUM_CLASSES = 80
CONF_T = 0.05
IOU_T = 0.5
MAX_PER_CLASS = 100
MAX_DET = 100

N_ANCHORS = 49104
LANES = 128
ROWS = 384
CHUNK_ROWS = 48
CHUNK = CHUNK_ROWS * LANES
N_CHUNKS = ROWS // CHUNK_ROWS
DEPTH = 8
NCAND = DEPTH * LANES
INT_MIN = np.int32(-(2 ** 31))
IDX_BIG = np.int32(2 ** 31 - 1)


def _to_key(s):
    i = jax.lax.bitcast_convert_type(s, jnp.int32)
    return jnp.where(i < 0, i ^ jnp.int32(0x7FFFFFFF), i)


def _from_key(k):
    i = jnp.where(k < 0, k ^ jnp.int32(0x7FFFFFFF), k)
    return jax.lax.bitcast_convert_type(i, jnp.float32)


def _select_kernel(pred_ref, keys_ref, idx_ref, cov_ref, ck_ref, ci_ref, ov_ref):
    j = pl.program_id(1)

    @pl.when(j == 0)
    def _init():
        ck_ref[...] = jnp.full((NCAND, NUM_CLASSES), INT_MIN, jnp.int32)
        ci_ref[...] = jnp.full((NCAND, NUM_CLASSES), IDX_BIG, jnp.int32)
        ov_ref[...] = jnp.full((8, NUM_CLASSES), INT_MIN, jnp.int32)

    lists_k = [ck_ref[pl.ds(d * LANES, LANES), :] for d in range(DEPTH)]
    lists_i = [ci_ref[pl.ds(d * LANES, LANES), :] for d in range(DEPTH)]
    ov0 = ov_ref[0:1, :]

    lane_anchor = jax.lax.broadcasted_iota(jnp.int32, (LANES, NUM_CLASSES), 0)

    def body(r, carry):
        ov = carry[0]
        lk = list(carry[1])
        li = list(carry[2])
        p = pred_ref[0, pl.ds(r * LANES, LANES), :]
        logits = p[:, 4:4 + NUM_CLASSES]
        s = jax.nn.sigmoid(logits)
        masked = jnp.where(s >= CONF_T, s, -1.0)
        v = _to_key(masked)
        grow = j * CHUNK_ROWS + r
        anchor = grow * LANES + lane_anchor
        v = jnp.where(anchor < N_ANCHORS, v, INT_MIN)
        vi = anchor
        for d in range(DEPTH):
            gt = v > lk[d]
            nk = jnp.where(gt, v, lk[d])
            ni = jnp.where(gt, vi, li[d])
            v = jnp.where(gt, lk[d], v)
            vi = jnp.where(gt, li[d], vi)
            lk[d] = nk
            li[d] = ni
        ov = jnp.maximum(ov, jnp.max(v, axis=0, keepdims=True))
        return (ov, tuple(lk), tuple(li))

    ov, lk, li = jax.lax.fori_loop(
        0, CHUNK_ROWS, body, (ov0, tuple(lists_k), tuple(lists_i)))

    for d in range(DEPTH):
        ck_ref[pl.ds(d * LANES, LANES), :] = lk[d]
        ci_ref[pl.ds(d * LANES, LANES), :] = li[d]
    ov_ref[0:1, :] = ov

    @pl.when(j == N_CHUNKS - 1)
    def _tail():
        def ebody(t, v100):
            kc = ck_ref[...]
            ic = ci_ref[...]
            mk = jnp.max(kc, axis=0, keepdims=True)
            eq = kc == mk
            mi = jnp.min(jnp.where(eq, ic, IDX_BIG), axis=0, keepdims=True)
            keys_ref[0, pl.ds(t, 1), :] = mk
            idx_ref[0, pl.ds(t, 1), :] = mi
            ck_ref[...] = jnp.where(eq & (ic == mi), INT_MIN, kc)
            return mk

        v100 = jax.lax.fori_loop(
            0, MAX_PER_CLASS, ebody,
            jnp.full((1, NUM_CLASSES), INT_MIN, jnp.int32))
        covered = (ov_ref[0:1, :] < v100).astype(jnp.int32)
        cov_ref[0] = jnp.broadcast_to(covered, (8, NUM_CLASSES))


def _select_stage(predictions):
    batch = predictions.shape[0]
    keys, idx, cov = pl.pallas_call(
        _select_kernel,
        grid=(batch, N_CHUNKS),
        in_specs=[
            pl.BlockSpec((1, CHUNK, 4 + NUM_CLASSES), lambda b, j: (b, j, 0)),
        ],
        out_specs=[
            pl.BlockSpec((1, 128, NUM_CLASSES), lambda b, j: (b, 0, 0)),
            pl.BlockSpec((1, 128, NUM_CLASSES), lambda b, j: (b, 0, 0)),
            pl.BlockSpec((1, 8, NUM_CLASSES), lambda b, j: (b, 0, 0)),
        ],
        out_shape=[
            jax.ShapeDtypeStruct((batch, 128, NUM_CLASSES), jnp.int32),
            jax.ShapeDtypeStruct((batch, 128, NUM_CLASSES), jnp.int32),
            jax.ShapeDtypeStruct((batch, 8, NUM_CLASSES), jnp.int32),
        ],
        scratch_shapes=[
            pltpu.VMEM((NCAND, NUM_CLASSES), jnp.int32),
            pltpu.VMEM((NCAND, NUM_CLASSES), jnp.int32),
            pltpu.VMEM((8, NUM_CLASSES), jnp.int32),
        ],
    )(predictions)
    keys = keys[:, :MAX_PER_CLASS]
    idx = idx[:, :MAX_PER_CLASS]
    covered = jnp.all(cov[:, 0, :] == 1)
    return keys, idx, covered


def _decode_corners(box_pred, anc):
    var = jnp.array([0.1, 0.1, 0.2, 0.2], dtype=jnp.float32)
    b = box_pred * var
    xy = b[..., :2] * anc[..., 2:] + anc[..., :2]
    wh = jnp.exp(b[..., 2:]) * anc[..., 2:]
    bb = jnp.concatenate([xy, wh], axis=-1)
    return jnp.concatenate([bb[..., :2] - bb[..., 2:] / 2.0,
                            bb[..., :2] + bb[..., 2:] / 2.0], axis=-1)


def _nms_scan(cand, top_s):
    ar = jnp.arange(MAX_PER_CLASS)

    def one_class(cand_c, s_c):
        def body(sc, i):
            valid = sc[i] > 0.0
            b = cand_c[i]
            x1 = jnp.maximum(b[0], cand_c[:, 0])
            y1 = jnp.maximum(b[1], cand_c[:, 1])
            x2 = jnp.minimum(b[2], cand_c[:, 2])
            y2 = jnp.minimum(b[3], cand_c[:, 3])
            inter = jnp.clip(x2 - x1, 0.0) * jnp.clip(y2 - y1, 0.0)
            a1 = (b[2] - b[0]) * (b[3] - b[1])
            a2 = (cand_c[:, 2] - cand_c[:, 0]) * (cand_c[:, 3] - cand_c[:, 1])
            iou = inter / (a1 + a2 - inter + 1e-8)
            suppress = (iou > IOU_T) & (ar > i) & valid
            return jnp.where(suppress, -1.0, sc), None

        s_final, _ = jax.lax.scan(body, s_c, ar)
        return s_final

    return jax.vmap(one_class)(cand, top_s)


def _merge(cand, s):
    flat_s = s.reshape(-1)
    flat_b = cand.reshape(-1, 4)
    fs, fi = jax.lax.top_k(flat_s, MAX_DET)
    fb = flat_b[fi]
    cls_ids = fi // MAX_PER_CLASS
    valid = fs > 0.0
    cls_ids = jnp.where(valid, cls_ids, -1)
    num = jnp.sum(valid.astype(jnp.int32))
    return fb, fs, cls_ids, num


def _finish_image(pred_b, anchors, top_keys, top_idx):
    top_s = _from_key(top_keys.T)
    top_i = top_idx.T
    cand = _decode_corners(pred_b[:, :4][top_i], anchors[top_i])
    s = _nms_scan(cand, top_s)
    return _merge(cand, s)


def _fallback_image(pred_b, anchors):
    scores = jax.nn.sigmoid(pred_b[:, 4:])
    boxes_c = _decode_corners(pred_b[:, :4], anchors)

    def one_class(s_c):
        sm = jnp.where(s_c >= CONF_T, s_c, -1.0)
        return jax.lax.top_k(sm, MAX_PER_CLASS)

    top_s, top_i = jax.vmap(one_class, in_axes=1)(scores)
    cand = boxes_c[top_i]
    s = _nms_scan(cand, top_s)
    return _merge(cand, s)


@functools.partial(jax.jit, static_argnums=())
def kernel(images, predictions, anchor_boxes):
    del images
    keys, idx, covered = _select_stage(predictions)

    def fast(_):
        return jax.vmap(_finish_image, in_axes=(0, None, 0, 0))(
            predictions, anchor_boxes, keys, idx)

    def slow(_):
        return jax.vmap(_fallback_image, in_axes=(0, None))(
            predictions, anchor_boxes)

    return jax.lax.cond(covered, fast, slow, operand=None)
